# baseline (device time: 16690 ns/iter reference)
import os

import jax
import jax.numpy as jnp

_NO_COMM = os.path.exists(os.path.join(
    os.path.dirname(__file__), "NO_COMM_FLAG"))
from jax import lax
from jax.experimental import pallas as pl
from jax.experimental.pallas import tpu as pltpu

N_DEV = 16
N_TOK = 512
D_IN = 256
D_OUT = 512
CAP = 12
BLK = 13


def kernel(x, router_W, route_idx, expert_W):
    def body(x_ref, rw_ref, idx_ref, w_ref, out_ref,
             g_ref, send_sems, recv_sems):
        my = lax.axis_index("i")

        if not _NO_COMM:
            bsem = pltpu.get_barrier_semaphore()
            for k in range(1, N_DEV):
                pl.semaphore_signal(
                    bsem, inc=1,
                    device_id=(lax.rem(my + k, N_DEV),),
                    device_id_type=pl.DeviceIdType.MESH,
                )

        idx = idx_ref[:, :]
        eids = my * 2 + lax.broadcasted_iota(jnp.int32, (1, 2), 1)
        onehot = (idx == eids).astype(jnp.float32)
        r_i = lax.broadcasted_iota(jnp.int32, (N_TOK, N_TOK), 0)
        c_i = lax.broadcasted_iota(jnp.int32, (N_TOK, N_TOK), 1)
        lower = (c_i <= r_i).astype(jnp.float32)
        pos = jnp.dot(lower, onehot, preferred_element_type=jnp.float32,
                      precision=lax.Precision.HIGHEST)
        keep = jnp.where((onehot > 0.5) & (pos < CAP + 0.5), 1.0, 0.0)
        slotv = keep[:, 0:1] * (pos[:, 0:1] - 1.0) \
            + keep[:, 1:2] * (pos[:, 1:2] + (CAP - 1.0))
        kept = keep[:, 0:1] + keep[:, 1:2]
        slotf = jnp.where(kept > 0.5, slotv, -1.0)
        cols = lax.broadcasted_iota(
            jnp.int32, (1, 2 * CAP), 1).astype(jnp.float32)
        cw = (slotf == cols).astype(jnp.float32)
        xsel = lax.dot_general(cw, x_ref[:, :], (((0,), (0,)), ((), ())),
                               preferred_element_type=jnp.float32)
        p0 = jnp.dot(xsel[0:CAP], w_ref[0],
                     preferred_element_type=jnp.float32)
        p1 = jnp.dot(xsel[CAP:2 * CAP], w_ref[1],
                     preferred_element_type=jnp.float32)
        iota1p = lax.broadcasted_iota(jnp.int32, (1, N_TOK), 1).astype(
            jnp.float32) + 1.0
        ridrow = jnp.dot(iota1p, cw, preferred_element_type=jnp.float32,
                         precision=lax.Precision.HIGHEST) - 1.0
        ridrow = jnp.concatenate(
            [ridrow, jnp.full((1, D_OUT - 2 * CAP), -1.0, jnp.float32)],
            axis=1)
        u0 = lax.bitcast_convert_type(
            p0.astype(jnp.bfloat16), jnp.uint16).astype(jnp.uint32)
        u1 = lax.bitcast_convert_type(
            p1.astype(jnp.bfloat16), jnp.uint16).astype(jnp.uint32)
        packed = lax.bitcast_convert_type(u0 | (u1 << 16), jnp.float32)
        g_ref[0, 0:CAP, :] = packed
        g_ref[0, CAP:BLK, :] = ridrow

        if not _NO_COMM:
            pl.semaphore_wait(bsem, N_DEV - 1)
        rdmas = []
        for k in () if _NO_COMM else range(1, N_DEV):
            tgt = lax.rem(my + k, N_DEV)
            rdma = pltpu.make_async_remote_copy(
                src_ref=g_ref.at[0],
                dst_ref=g_ref.at[N_DEV - k],
                send_sem=send_sems.at[k - 1],
                recv_sem=recv_sems.at[N_DEV - k - 1],
                device_id=(tgt,),
                device_id_type=pl.DeviceIdType.MESH,
            )
            rdma.start()
            rdmas.append(rdma)

        toks = lax.broadcasted_iota(jnp.int32, (N_TOK, 1), 0).astype(
            jnp.float32)
        pieces_scat = [None] * N_DEV
        pieces_pay = [None] * N_DEV
        def process(s):
            blk = g_ref[s, :, :]
            u = lax.bitcast_convert_type(blk[0:CAP, :], jnp.uint32)
            lo = lax.bitcast_convert_type(
                (u & 0xFFFF).astype(jnp.uint16), jnp.bfloat16)
            hi = lax.bitcast_convert_type(
                (u >> 16).astype(jnp.uint16), jnp.bfloat16)
            pieces_pay[s] = jnp.concatenate([lo, hi], axis=0)
            rid_s = blk[CAP:CAP + 1, 0:2 * CAP]
            pieces_scat[s] = (toks == rid_s).astype(jnp.bfloat16)
        process(0)
        for k in range(1, N_DEV):
            if not _NO_COMM:
                rdmas[k - 1].wait_recv()
            process(N_DEV - k)
        for rdma in rdmas:
            rdma.wait_send()
        scat = jnp.concatenate(pieces_scat, axis=1)
        pay = jnp.concatenate(pieces_pay, axis=0)
        out_ref[:, :] = jnp.dot(scat, pay,
                                preferred_element_type=jnp.float32)

    return pl.pallas_call(
        body,
        out_shape=jax.ShapeDtypeStruct((N_TOK, D_OUT), jnp.float32),
        in_specs=[pl.BlockSpec(memory_space=pltpu.VMEM)] * 4,
        out_specs=pl.BlockSpec(memory_space=pltpu.VMEM),
        scratch_shapes=[
            pltpu.VMEM((N_DEV, BLK, D_OUT), jnp.float32),
            pltpu.SemaphoreType.DMA((N_DEV - 1,)),
            pltpu.SemaphoreType.DMA((N_DEV - 1,)),
        ],
        compiler_params=(None if _NO_COMM
                         else pltpu.CompilerParams(collective_id=0)),
    )(x, router_W, route_idx, expert_W)


# device time: 16531 ns/iter; 1.0096x vs baseline; 1.0096x over previous
import os

import jax
import jax.numpy as jnp

_NO_COMM = os.path.exists(os.path.join(
    os.path.dirname(__file__), "NO_COMM_FLAG"))
from jax import lax
from jax.experimental import pallas as pl
from jax.experimental.pallas import tpu as pltpu

N_DEV = 16
N_TOK = 512
D_IN = 256
D_OUT = 512
CAP = 12
BLK = 13


def kernel(x, router_W, route_idx, expert_W):
    def body(x_ref, rw_ref, idx_ref, w_ref, out_ref,
             g_ref, send_sems, recv_sems):
        my = lax.axis_index("i")

        if not _NO_COMM:
            bsem = pltpu.get_barrier_semaphore()
            for k in range(1, N_DEV):
                pl.semaphore_signal(
                    bsem, inc=1,
                    device_id=(lax.rem(my + k, N_DEV),),
                    device_id_type=pl.DeviceIdType.MESH,
                )

        idx = idx_ref[:, :]
        eids = my * 2 + lax.broadcasted_iota(jnp.int32, (1, 2), 1)
        onehot = (idx == eids).astype(jnp.float32)
        r_i = lax.broadcasted_iota(jnp.int32, (N_TOK, N_TOK), 0)
        c_i = lax.broadcasted_iota(jnp.int32, (N_TOK, N_TOK), 1)
        lower = (c_i <= r_i).astype(jnp.float32)
        pos = jnp.dot(lower, onehot, preferred_element_type=jnp.float32,
                      precision=lax.Precision.HIGHEST)
        keep = jnp.where((onehot > 0.5) & (pos < CAP + 0.5), 1.0, 0.0)
        slotv = keep[:, 0:1] * (pos[:, 0:1] - 1.0) \
            + keep[:, 1:2] * (pos[:, 1:2] + (CAP - 1.0))
        kept = keep[:, 0:1] + keep[:, 1:2]
        slotf = jnp.where(kept > 0.5, slotv, -1.0)
        cols = lax.broadcasted_iota(
            jnp.int32, (1, 2 * CAP), 1).astype(jnp.float32)
        cw = (slotf == cols).astype(jnp.float32)
        xsel = lax.dot_general(cw, x_ref[:, :], (((0,), (0,)), ((), ())),
                               preferred_element_type=jnp.float32)
        p0 = jnp.dot(xsel[0:CAP], w_ref[0],
                     preferred_element_type=jnp.float32)
        p1 = jnp.dot(xsel[CAP:2 * CAP], w_ref[1],
                     preferred_element_type=jnp.float32)
        iota1p = lax.broadcasted_iota(jnp.int32, (1, N_TOK), 1).astype(
            jnp.float32) + 1.0
        ridrow = jnp.dot(iota1p, cw, preferred_element_type=jnp.float32,
                         precision=lax.Precision.HIGHEST) - 1.0
        ridrow = jnp.concatenate(
            [ridrow, jnp.full((1, D_OUT - 2 * CAP), -1.0, jnp.float32)],
            axis=1)
        u0 = lax.bitcast_convert_type(
            p0.astype(jnp.bfloat16), jnp.uint16).astype(jnp.uint32)
        u1 = lax.bitcast_convert_type(
            p1.astype(jnp.bfloat16), jnp.uint16).astype(jnp.uint32)
        packed = lax.bitcast_convert_type(u0 | (u1 << 16), jnp.float32)
        g_ref[0, 0:CAP, :] = packed
        g_ref[0, CAP:BLK, :] = ridrow

        if not _NO_COMM:
            pl.semaphore_wait(bsem, N_DEV - 1)
        rdmas = []
        for k in () if _NO_COMM else range(1, N_DEV):
            tgt = lax.rem(my + k, N_DEV)
            rdma = pltpu.make_async_remote_copy(
                src_ref=g_ref.at[0],
                dst_ref=g_ref.at[N_DEV - k],
                send_sem=send_sems.at[k - 1],
                recv_sem=recv_sems.at[N_DEV - k - 1],
                device_id=(tgt,),
                device_id_type=pl.DeviceIdType.MESH,
            )
            rdma.start()
            rdmas.append(rdma)

        toks = lax.broadcasted_iota(jnp.int32, (N_TOK, 1), 0).astype(
            jnp.float32)
        pieces_rid = [None] * N_DEV
        pieces_pay = [None] * N_DEV
        def process(s):
            blk = g_ref[s, :, :]
            u = lax.bitcast_convert_type(blk[0:CAP, :], jnp.uint32)
            lo = lax.bitcast_convert_type(
                (u & 0xFFFF).astype(jnp.uint16), jnp.bfloat16)
            hi = lax.bitcast_convert_type(
                (u >> 16).astype(jnp.uint16), jnp.bfloat16)
            pieces_pay[s] = jnp.concatenate([lo, hi], axis=0)
            pieces_rid[s] = blk[CAP:CAP + 1, 0:2 * CAP]
        process(0)
        for k in range(1, N_DEV):
            if not _NO_COMM:
                rdmas[k - 1].wait_recv()
            process(N_DEV - k)
        for rdma in rdmas:
            rdma.wait_send()
        rid_all = jnp.concatenate(pieces_rid, axis=1)
        scat = (toks == rid_all).astype(jnp.bfloat16)
        pay = jnp.concatenate(pieces_pay, axis=0)
        out_ref[:, :] = jnp.dot(scat, pay,
                                preferred_element_type=jnp.float32)

    return pl.pallas_call(
        body,
        out_shape=jax.ShapeDtypeStruct((N_TOK, D_OUT), jnp.float32),
        in_specs=[pl.BlockSpec(memory_space=pltpu.VMEM)] * 4,
        out_specs=pl.BlockSpec(memory_space=pltpu.VMEM),
        scratch_shapes=[
            pltpu.VMEM((N_DEV, BLK, D_OUT), jnp.float32),
            pltpu.SemaphoreType.DMA((N_DEV - 1,)),
            pltpu.SemaphoreType.DMA((N_DEV - 1,)),
        ],
        compiler_params=(None if _NO_COMM
                         else pltpu.CompilerParams(collective_id=0)),
    )(x, router_W, route_idx, expert_W)


# device time: 16187 ns/iter; 1.0311x vs baseline; 1.0213x over previous
import jax
import jax.numpy as jnp
from jax import lax
from jax.experimental import pallas as pl
from jax.experimental.pallas import tpu as pltpu

N_DEV = 16
N_TOK = 512
D_IN = 256
D_OUT = 512
CAP = 12
BLK = 13


def kernel(x, router_W, route_idx, expert_W):
    def body(x_ref, rw_ref, idx_ref, w_ref, out_ref,
             g_ref, send_sems, recv_sems):
        my = lax.axis_index("i")

        bsem = pltpu.get_barrier_semaphore()
        for k in range(1, N_DEV):
            pl.semaphore_signal(
                bsem, inc=1,
                device_id=(lax.rem(my + k, N_DEV),),
                device_id_type=pl.DeviceIdType.MESH,
            )

        idx = idx_ref[:, :]
        eids = my * 2 + lax.broadcasted_iota(jnp.int32, (1, 2), 1)
        onehot = (idx == eids).astype(jnp.float32)
        r_i = lax.broadcasted_iota(jnp.int32, (N_TOK, N_TOK), 0)
        c_i = lax.broadcasted_iota(jnp.int32, (N_TOK, N_TOK), 1)
        lower = (c_i <= r_i).astype(jnp.float32)
        pos = jnp.dot(lower, onehot, preferred_element_type=jnp.float32)
        keep = jnp.where((onehot > 0.5) & (pos < CAP + 0.5), 1.0, 0.0)
        slotv = keep[:, 0:1] * (pos[:, 0:1] - 1.0) \
            + keep[:, 1:2] * (pos[:, 1:2] + (CAP - 1.0))
        kept = keep[:, 0:1] + keep[:, 1:2]
        slotf = jnp.where(kept > 0.5, slotv, -1.0)
        cols = lax.broadcasted_iota(
            jnp.int32, (1, 2 * CAP), 1).astype(jnp.float32)
        cw = (slotf == cols).astype(jnp.float32)
        xsel = lax.dot_general(cw, x_ref[:, :], (((0,), (0,)), ((), ())),
                               preferred_element_type=jnp.float32)
        p0 = jnp.dot(xsel[0:CAP], w_ref[0],
                     preferred_element_type=jnp.float32)
        p1 = jnp.dot(xsel[CAP:2 * CAP], w_ref[1],
                     preferred_element_type=jnp.float32)
        iota1p = lax.broadcasted_iota(jnp.int32, (1, N_TOK), 1).astype(
            jnp.float32) + 1.0
        ridrow = jnp.dot(iota1p, cw, preferred_element_type=jnp.float32,
                         precision=lax.Precision.HIGHEST) - 1.0
        ridrow = jnp.concatenate(
            [ridrow, jnp.full((1, D_OUT - 2 * CAP), -1.0, jnp.float32)],
            axis=1)
        u0 = lax.bitcast_convert_type(
            p0.astype(jnp.bfloat16), jnp.uint16).astype(jnp.uint32)
        u1 = lax.bitcast_convert_type(
            p1.astype(jnp.bfloat16), jnp.uint16).astype(jnp.uint32)
        packed = lax.bitcast_convert_type(u0 | (u1 << 16), jnp.float32)
        g_ref[0, 0:CAP, :] = packed
        g_ref[0, CAP:BLK, :] = ridrow

        pl.semaphore_wait(bsem, N_DEV - 1)
        rdmas = []
        for k in range(1, N_DEV):
            tgt = lax.rem(my + k, N_DEV)
            rdma = pltpu.make_async_remote_copy(
                src_ref=g_ref.at[0],
                dst_ref=g_ref.at[N_DEV - k],
                send_sem=send_sems.at[k - 1],
                recv_sem=recv_sems.at[N_DEV - k - 1],
                device_id=(tgt,),
                device_id_type=pl.DeviceIdType.MESH,
            )
            rdma.start()
            rdmas.append(rdma)

        toks = lax.broadcasted_iota(jnp.int32, (N_TOK, 1), 0).astype(
            jnp.float32)
        pieces_rid = [None] * N_DEV
        pieces_pay = [None] * N_DEV
        def process(s):
            blk = g_ref[s, :, :]
            u = lax.bitcast_convert_type(blk[0:CAP, :], jnp.uint32)
            lo = lax.bitcast_convert_type(
                (u & 0xFFFF).astype(jnp.uint16), jnp.bfloat16)
            hi = lax.bitcast_convert_type(
                (u >> 16).astype(jnp.uint16), jnp.bfloat16)
            pieces_pay[s] = jnp.concatenate([lo, hi], axis=0)
            pieces_rid[s] = blk[CAP:CAP + 1, 0:2 * CAP]
        process(0)
        for k in range(1, N_DEV):
            rdmas[k - 1].wait_recv()
            process(N_DEV - k)
        for rdma in rdmas:
            rdma.wait_send()
        rid_all = jnp.concatenate(pieces_rid, axis=1)
        scat = (toks == rid_all).astype(jnp.bfloat16)
        pay = jnp.concatenate(pieces_pay, axis=0)
        out_ref[:, :] = jnp.dot(scat, pay,
                                preferred_element_type=jnp.float32)

    return pl.pallas_call(
        body,
        out_shape=jax.ShapeDtypeStruct((N_TOK, D_OUT), jnp.float32),
        in_specs=[pl.BlockSpec(memory_space=pltpu.VMEM)] * 4,
        out_specs=pl.BlockSpec(memory_space=pltpu.VMEM),
        scratch_shapes=[
            pltpu.VMEM((N_DEV, BLK, D_OUT), jnp.float32),
            pltpu.SemaphoreType.DMA((N_DEV - 1,)),
            pltpu.SemaphoreType.DMA((N_DEV - 1,)),
        ],
        compiler_params=pltpu.CompilerParams(collective_id=0),
    )(x, router_W, route_idx, expert_W)
